# Initial kernel scaffold; baseline (speedup 1.0000x reference)
import functools

import jax
import jax.numpy as jnp
from jax import lax
from jax.experimental import pallas as pl
from jax.experimental.pallas import tpu as pltpu
from jax.experimental.pallas import tpu_sc as plsc

DIM = 128
BATCH = 1024
SEQ = 200
HALF = SEQ // 2
NCORES = 2
NSUB = 16
NWORKERS = NCORES * NSUB
ROWS_PER_WORKER = BATCH // NWORKERS
LANES = 16
NSLICE = DIM // LANES


def _rsqrt16(v):
    i = plsc.bitcast(v, jnp.int32)
    i = jnp.int32(0x5F3759DF) - (i >> 1)
    y = plsc.bitcast(i, jnp.float32)
    for _ in range(3):
        y = y * (1.5 - 0.5 * v * y * y)
    return y


def _lane_total(v, perms):
    for p in perms:
        v = v + jnp.take(v, p, mode="promise_in_bounds")
    return v


_mesh = plsc.VectorSubcoreMesh(core_axis_name="c", subcore_axis_name="s")


@functools.partial(
    pl.kernel,
    out_type=jax.ShapeDtypeStruct((BATCH, SEQ, DIM), jnp.float32),
    mesh=_mesh,
    scratch_types=[
        pltpu.VMEM((2, HALF), jnp.int32),
        pltpu.VMEM((SEQ, DIM), jnp.float32),
        pltpu.VMEM((SEQ, DIM), jnp.float32),
        pltpu.VMEM((DIM,), jnp.float32),
        pltpu.VMEM((DIM,), jnp.float32),
        pltpu.SemaphoreType.DMA,
    ],
)
def _emb_ln_kernel(ids_hbm, word_hbm, pos_hbm, gamma_hbm, beta_hbm, out_hbm,
                   idx_v, rows_v, pos_v, gamma_v, beta_v, sem):
    wid = lax.axis_index("s") * NCORES + lax.axis_index("c")

    pltpu.sync_copy(pos_hbm.at[pl.ds(0, SEQ)], pos_v)
    pltpu.sync_copy(gamma_hbm, gamma_v)
    pltpu.sync_copy(beta_hbm, beta_v)

    iota = lax.iota(jnp.int32, LANES)
    perms = [iota ^ k for k in (1, 2, 4, 8)]
    gs = [gamma_v[pl.ds(LANES * j, LANES)] for j in range(NSLICE)]
    bs = [beta_v[pl.ds(LANES * j, LANES)] for j in range(NSLICE)]

    def batch_body(i, carry):
        b = wid * ROWS_PER_WORKER + i
        pltpu.sync_copy(ids_hbm.at[b], idx_v)
        cp0 = pltpu.async_copy(word_hbm.at[idx_v.at[0]],
                               rows_v.at[pl.ds(0, HALF)], sem)
        cp1 = pltpu.async_copy(word_hbm.at[idx_v.at[1]],
                               rows_v.at[pl.ds(HALF, HALF)], sem)
        cp0.wait()
        cp1.wait()

        def row_body(r, rcarry):
            xs = [rows_v[r, pl.ds(LANES * j, LANES)]
                  + pos_v[r, pl.ds(LANES * j, LANES)]
                  for j in range(NSLICE)]
            s = xs[0]
            q = xs[0] * xs[0]
            for j in range(1, NSLICE):
                s = s + xs[j]
                q = q + xs[j] * xs[j]
            s = _lane_total(s, perms)
            q = _lane_total(q, perms)
            mean = s * (1.0 / DIM)
            var = jnp.maximum(q * (1.0 / DIM) - mean * mean, 0.0)
            rstd = _rsqrt16(var + 1e-12)
            mr = mean * rstd
            for j in range(NSLICE):
                t = xs[j] * rstd - mr
                rows_v[r, pl.ds(LANES * j, LANES)] = t * gs[j] + bs[j]
            return rcarry

        lax.fori_loop(0, SEQ, row_body, 0)
        pltpu.sync_copy(rows_v, out_hbm.at[b])
        return carry

    lax.fori_loop(0, ROWS_PER_WORKER, batch_body, 0)


def kernel(input_ids, word_table, pos_table, gamma, beta):
    ids = input_ids.astype(jnp.int32).reshape(BATCH, 2, HALF)
    return _emb_ln_kernel(ids, word_table, pos_table, gamma, beta)


# speedup vs baseline: 3.7036x; 3.7036x over previous
import functools

import jax
import jax.numpy as jnp
from jax import lax
from jax.experimental import pallas as pl
from jax.experimental.pallas import tpu as pltpu
from jax.experimental.pallas import tpu_sc as plsc

DIM = 128
BATCH = 1024
SEQ = 200
HALF = SEQ // 2
NCORES = 2
NSUB = 16
NWORKERS = NCORES * NSUB
ROWS_PER_WORKER = BATCH // NWORKERS
LANES = 16
NSLICE = DIM // LANES


def _rsqrt16(v):
    i = lax.bitcast_convert_type(v, jnp.int32)
    i = jnp.int32(0x5F3759DF) - (i >> 1)
    y = lax.bitcast_convert_type(i, jnp.float32)
    for _ in range(3):
        y = y * (1.5 - 0.5 * v * y * y)
    return y


_GATHER_DNUMS = lax.GatherDimensionNumbers(
    offset_dims=(), collapsed_slice_dims=(0,), start_index_map=(0,))


def _shuffle(v, p2):
    return lax.gather(v, p2, _GATHER_DNUMS, (1,),
                      mode=lax.GatherScatterMode.PROMISE_IN_BOUNDS)


def _lane_total(v, perms):
    for p2 in perms:
        v = v + _shuffle(v, p2)
    return v


_mesh = plsc.VectorSubcoreMesh(core_axis_name="c", subcore_axis_name="s")


@functools.partial(
    pl.kernel,
    out_type=jax.ShapeDtypeStruct((BATCH, SEQ, DIM), jnp.float32),
    mesh=_mesh,
    scratch_types=[
        pltpu.VMEM((2, HALF), jnp.int32),
        pltpu.VMEM((SEQ, DIM), jnp.float32),
        pltpu.VMEM((SEQ, DIM), jnp.float32),
        pltpu.VMEM((DIM,), jnp.float32),
        pltpu.VMEM((DIM,), jnp.float32),
        pltpu.SemaphoreType.DMA,
    ],
)
def _emb_ln_kernel(ids_hbm, word_hbm, pos_hbm, gamma_hbm, beta_hbm, out_hbm,
                   idx_v, rows_v, pos_v, gamma_v, beta_v, sem):
    wid = lax.axis_index("s") * NCORES + lax.axis_index("c")

    pltpu.sync_copy(pos_hbm.at[pl.ds(0, SEQ)], pos_v)
    pltpu.sync_copy(gamma_hbm, gamma_v)
    pltpu.sync_copy(beta_hbm, beta_v)

    iota = lax.iota(jnp.int32, LANES)
    perms = [(iota ^ k).reshape(LANES, 1) for k in (1, 2, 4, 8)]
    gs = [gamma_v[pl.ds(LANES * j, LANES)] for j in range(NSLICE)]
    bs = [beta_v[pl.ds(LANES * j, LANES)] for j in range(NSLICE)]

    def batch_body(i, carry):
        b = wid * ROWS_PER_WORKER + i
        pltpu.sync_copy(ids_hbm.at[b], idx_v)
        cp0 = pltpu.async_copy(word_hbm.at[idx_v.at[0]],
                               rows_v.at[pl.ds(0, HALF)], sem)
        cp1 = pltpu.async_copy(word_hbm.at[idx_v.at[1]],
                               rows_v.at[pl.ds(HALF, HALF)], sem)
        cp0.wait()
        cp1.wait()

        def row_body(r, rcarry):
            xs = [rows_v[r, pl.ds(LANES * j, LANES)]
                  + pos_v[r, pl.ds(LANES * j, LANES)]
                  for j in range(NSLICE)]
            s = xs[0]
            q = xs[0] * xs[0]
            for j in range(1, NSLICE):
                s = s + xs[j]
                q = q + xs[j] * xs[j]
            s = _lane_total(s, perms)
            q = _lane_total(q, perms)
            mean = s * (1.0 / DIM)
            var = jnp.maximum(q * (1.0 / DIM) - mean * mean, 0.0)
            rstd = _rsqrt16(var + 1e-12)
            mr = mean * rstd
            for j in range(NSLICE):
                t = xs[j] * rstd - mr
                rows_v[r, pl.ds(LANES * j, LANES)] = t * gs[j] + bs[j]
            return rcarry

        lax.fori_loop(0, SEQ, row_body, 0)
        pltpu.sync_copy(rows_v, out_hbm.at[b])
        return carry

    lax.fori_loop(0, ROWS_PER_WORKER, batch_body, 0)


def kernel(input_ids, word_table, pos_table, gamma, beta):
    ids = input_ids.astype(jnp.int32).reshape(BATCH, 2, HALF)
    return _emb_ln_kernel(ids, word_table, pos_table, gamma, beta)


# speedup vs baseline: 4.4410x; 1.1991x over previous
import functools

import jax
import jax.numpy as jnp
from jax import lax
from jax.experimental import pallas as pl
from jax.experimental.pallas import tpu as pltpu
from jax.experimental.pallas import tpu_sc as plsc

DIM = 128
BATCH = 1024
SEQ = 200
HALF = SEQ // 2
NCORES = 2
NSUB = 16
NWORKERS = NCORES * NSUB
ROWS_PER_WORKER = BATCH // NWORKERS
LANES = 16
NSLICE = DIM // LANES


def _rsqrt16(v):
    i = lax.bitcast_convert_type(v, jnp.int32)
    i = jnp.int32(0x5F3759DF) - (i >> 1)
    y = lax.bitcast_convert_type(i, jnp.float32)
    for _ in range(3):
        y = y * (1.5 - 0.5 * v * y * y)
    return y


_GATHER_DNUMS = lax.GatherDimensionNumbers(
    offset_dims=(), collapsed_slice_dims=(0,), start_index_map=(0,))


def _shuffle(v, p2):
    return lax.gather(v, p2, _GATHER_DNUMS, (1,),
                      mode=lax.GatherScatterMode.PROMISE_IN_BOUNDS)


def _lane_total(v, perms):
    for p2 in perms:
        v = v + _shuffle(v, p2)
    return v


_mesh = plsc.VectorSubcoreMesh(core_axis_name="c", subcore_axis_name="s")


@functools.partial(
    pl.kernel,
    out_type=jax.ShapeDtypeStruct((BATCH, SEQ, DIM), jnp.float32),
    mesh=_mesh,
    scratch_types=[
        pltpu.VMEM((2, HALF), jnp.int32),
        pltpu.VMEM((2, HALF), jnp.int32),
        pltpu.VMEM((SEQ, DIM), jnp.float32),
        pltpu.VMEM((SEQ, DIM), jnp.float32),
        pltpu.VMEM((SEQ, DIM), jnp.float32),
        pltpu.VMEM((DIM,), jnp.float32),
        pltpu.VMEM((DIM,), jnp.float32),
        pltpu.SemaphoreType.DMA,
        pltpu.SemaphoreType.DMA,
        pltpu.SemaphoreType.DMA,
        pltpu.SemaphoreType.DMA,
    ],
)
def _emb_ln_kernel(ids_hbm, word_hbm, pos_hbm, gamma_hbm, beta_hbm, out_hbm,
                   idx0, idx1, buf0, buf1, pos_v, gamma_v, beta_v,
                   semg0, semg1, sems0, sems1):
    wid = lax.axis_index("s") * NCORES + lax.axis_index("c")
    base = wid * ROWS_PER_WORKER

    pltpu.sync_copy(pos_hbm.at[pl.ds(0, SEQ)], pos_v)
    pltpu.sync_copy(gamma_hbm, gamma_v)
    pltpu.sync_copy(beta_hbm, beta_v)

    iota = lax.iota(jnp.int32, LANES)
    perms = [(iota ^ k).reshape(LANES, 1) for k in (1, 2, 4, 8)]
    gs = [gamma_v[pl.ds(LANES * j, LANES)] for j in range(NSLICE)]
    bs = [beta_v[pl.ds(LANES * j, LANES)] for j in range(NSLICE)]

    def start_gather(idx_v, buf, semg):
        pltpu.async_copy(word_hbm.at[idx_v.at[0]],
                         buf.at[pl.ds(0, HALF)], semg)
        pltpu.async_copy(word_hbm.at[idx_v.at[1]],
                         buf.at[pl.ds(HALF, HALF)], semg)

    def wait_gather(idx_v, buf, semg):
        pltpu.make_async_copy(word_hbm.at[idx_v.at[0]],
                              buf.at[pl.ds(0, HALF)], semg).wait()
        pltpu.make_async_copy(word_hbm.at[idx_v.at[1]],
                              buf.at[pl.ds(HALF, HALF)], semg).wait()

    def compute(buf):
        def row_body(r, rcarry):
            xs = [buf[r, pl.ds(LANES * j, LANES)]
                  + pos_v[r, pl.ds(LANES * j, LANES)]
                  for j in range(NSLICE)]
            s = xs[0]
            q = xs[0] * xs[0]
            for j in range(1, NSLICE):
                s = s + xs[j]
                q = q + xs[j] * xs[j]
            s = _lane_total(s, perms)
            q = _lane_total(q, perms)
            mean = s * (1.0 / DIM)
            var = jnp.maximum(q * (1.0 / DIM) - mean * mean, 0.0)
            rstd = _rsqrt16(var + 1e-12)
            mr = mean * rstd
            for j in range(NSLICE):
                t = xs[j] * rstd - mr
                buf[r, pl.ds(LANES * j, LANES)] = t * gs[j] + bs[j]
            return rcarry

        lax.fori_loop(0, SEQ, row_body, 0)

    def step(i, idxA, bufA, semgA, semsA, idxB, bufB, semgB, semsB):
        b = base + i

        @pl.when(i + 1 < ROWS_PER_WORKER)
        def _():
            pltpu.sync_copy(ids_hbm.at[b + 1], idxB)

            @pl.when(i >= 1)
            def _():
                pltpu.make_async_copy(bufB, out_hbm.at[b - 1], semsB).wait()

            start_gather(idxB, bufB, semgB)

        wait_gather(idxA, bufA, semgA)
        compute(bufA)
        pltpu.async_copy(bufA, out_hbm.at[b], semsA)

    pltpu.sync_copy(ids_hbm.at[base], idx0)
    start_gather(idx0, buf0, semg0)

    def pair_body(k, carry):
        i = 2 * k
        step(i, idx0, buf0, semg0, sems0, idx1, buf1, semg1, sems1)
        step(i + 1, idx1, buf1, semg1, sems1, idx0, buf0, semg0, sems0)
        return carry

    lax.fori_loop(0, ROWS_PER_WORKER // 2, pair_body, 0)

    pltpu.make_async_copy(buf0, out_hbm.at[base + ROWS_PER_WORKER - 2],
                          sems0).wait()
    pltpu.make_async_copy(buf1, out_hbm.at[base + ROWS_PER_WORKER - 1],
                          sems1).wait()


def kernel(input_ids, word_table, pos_table, gamma, beta):
    ids = input_ids.astype(jnp.int32).reshape(BATCH, 2, HALF)
    return _emb_ln_kernel(ids, word_table, pos_table, gamma, beta)
